# Initial kernel scaffold; baseline (speedup 1.0000x reference)
import jax
import jax.numpy as jnp
from jax.experimental import pallas as pl


def kernel(x, edge_index, W0, b0, W1, b1, W2, b2, M0, mb0, M1, mb1, M2, mb2):
    raise NotImplementedError("write your pallas kernel here")


# speedup vs baseline: 3.7688x; 3.7688x over previous
import jax
import jax.numpy as jnp
from jax import lax
from jax.experimental import pallas as pl
from jax.experimental.pallas import tpu as pltpu
from jax.experimental.pallas import tpu_sc as plsc

N = 10000
D = 128
E = 320000
NC, NS = 2, 16
NW = NC * NS
B = 128
CH = -(-E // (NW * B))
EPAD = NW * CH * B
SB = 5
STRIPE = SB * B
NP = NS * STRIPE

_mesh = plsc.VectorSubcoreMesh(
    core_axis_name="c", subcore_axis_name="s", num_cores=NC, num_subcores=NS
)
_f32 = jnp.float32


def _zero_stripe(sh, buf, s):
    for k in range(SB):
        pltpu.sync_copy(buf, sh.at[pl.ds(s * STRIPE + k * B, B)])


def _copy_out_stripe(sh, buf, out, s):
    for k in range(SB):
        so = pl.ds(s * STRIPE + k * B, B)
        pltpu.sync_copy(sh.at[so], buf)
        pltpu.sync_copy(buf, out.at[so])


def _segsum_body(h, srcp, dstp, zrow, outA0, outA1,
                 sidx, didx, rows, A_sh, sem):
    c = lax.axis_index("c")
    s = lax.axis_index("s")
    wid = c * NS + s

    pltpu.sync_copy(zrow, rows)
    _zero_stripe(A_sh, rows, s)
    plsc.subcore_barrier()

    base0 = wid * (CH * B)

    def chunk(ci, carry):
        base = base0 + ci * B
        pltpu.sync_copy(srcp.at[pl.ds(base, B)], sidx)
        pltpu.sync_copy(dstp.at[pl.ds(base, B)], didx)
        pltpu.async_copy(h.at[sidx], rows, sem).wait()
        pltpu.sync_copy(rows, A_sh.at[didx], add=True)
        return carry

    lax.fori_loop(0, CH, chunk, 0)
    plsc.subcore_barrier()

    @pl.when(c == 0)
    def _():
        _copy_out_stripe(A_sh, rows, outA0, s)

    @pl.when(c == 1)
    def _():
        _copy_out_stripe(A_sh, rows, outA1, s)


_segsum = pl.kernel(
    _segsum_body,
    out_type=[jax.ShapeDtypeStruct((NP, D), _f32),
              jax.ShapeDtypeStruct((NP, D), _f32)],
    mesh=_mesh,
    scratch_types=[
        pltpu.VMEM((B,), jnp.int32),
        pltpu.VMEM((B,), jnp.int32),
        pltpu.VMEM((B, D), _f32),
        pltpu.VMEM_SHARED((NP, D), _f32),
        pltpu.SemaphoreType.DMA,
    ],
)


def _deg_body(dstp, zrow, onesr, outD0, outD1, didx, rows, G_sh):
    c = lax.axis_index("c")
    s = lax.axis_index("s")
    wid = c * NS + s

    pltpu.sync_copy(zrow, rows)
    _zero_stripe(G_sh, rows, s)
    plsc.subcore_barrier()

    pltpu.sync_copy(onesr, rows)
    base0 = wid * (CH * B)

    def chunk(ci, carry):
        base = base0 + ci * B
        pltpu.sync_copy(dstp.at[pl.ds(base, B)], didx)
        pltpu.sync_copy(rows, G_sh.at[didx], add=True)
        return carry

    lax.fori_loop(0, CH, chunk, 0)
    plsc.subcore_barrier()

    @pl.when(c == 0)
    def _():
        _copy_out_stripe(G_sh, rows, outD0, s)

    @pl.when(c == 1)
    def _():
        _copy_out_stripe(G_sh, rows, outD1, s)


_deg = pl.kernel(
    _deg_body,
    out_type=[jax.ShapeDtypeStruct((NP, D), _f32),
              jax.ShapeDtypeStruct((NP, D), _f32)],
    mesh=_mesh,
    scratch_types=[
        pltpu.VMEM((B,), jnp.int32),
        pltpu.VMEM((B, D), _f32),
        pltpu.VMEM_SHARED((NP, D), _f32),
    ],
)


_RB = 2000


def _conv_body(a0, a1, d0, d1, w, bb, o):
    acc = jnp.dot(a0[...] + a1[...], w[...], preferred_element_type=_f32)
    o[...] = jnp.maximum(acc + (d0[...] + d1[...]) * bb[...], 0.0)


def _conv_tc(A0, A1, DG0, DG1, W, b):
    blk = lambda i: (i, 0)
    fixed = lambda i: (0, 0)
    return pl.pallas_call(
        _conv_body,
        grid=(N // _RB,),
        in_specs=[
            pl.BlockSpec((_RB, D), blk),
            pl.BlockSpec((_RB, D), blk),
            pl.BlockSpec((_RB, D), blk),
            pl.BlockSpec((_RB, D), blk),
            pl.BlockSpec((D, D), fixed),
            pl.BlockSpec((1, D), fixed),
        ],
        out_specs=pl.BlockSpec((_RB, D), blk),
        out_shape=jax.ShapeDtypeStruct((N, D), _f32),
    )(A0, A1, DG0, DG1, W, b.reshape(1, D))


def _final_body(a0, a1, d0, d1, w, bb, m0, c0, m1, c1, m2, c2, o):
    h = jnp.maximum(
        jnp.dot(a0[...] + a1[...], w[...], preferred_element_type=_f32)
        + (d0[...] + d1[...]) * bb[...], 0.0)
    y = jnp.maximum(jnp.dot(h, m0[...], preferred_element_type=_f32) + c0[...], 0.0)
    y = jnp.maximum(jnp.dot(y, m1[...], preferred_element_type=_f32) + c1[...], 0.0)
    o[...] = jnp.dot(y, m2[...], preferred_element_type=_f32) + c2[...]


def _final_tc(A0, A1, DG0, DG1, W, b, M0, mb0, M1, mb1, M2, mb2):
    blk = lambda i: (i, 0)
    fixed = lambda i: (0, 0)
    return pl.pallas_call(
        _final_body,
        grid=(N // _RB,),
        in_specs=[
            pl.BlockSpec((_RB, D), blk),
            pl.BlockSpec((_RB, D), blk),
            pl.BlockSpec((_RB, D), blk),
            pl.BlockSpec((_RB, D), blk),
            pl.BlockSpec((D, D), fixed),
            pl.BlockSpec((1, D), fixed),
            pl.BlockSpec((D, D), fixed),
            pl.BlockSpec((1, D), fixed),
            pl.BlockSpec((D, D), fixed),
            pl.BlockSpec((1, D), fixed),
            pl.BlockSpec((D, 1), fixed),
            pl.BlockSpec((1, 1), fixed),
        ],
        out_specs=pl.BlockSpec((_RB, 1), blk),
        out_shape=jax.ShapeDtypeStruct((N, 1), _f32),
    )(A0, A1, DG0, DG1, W, b.reshape(1, D),
      M0, mb0.reshape(1, D), M1, mb1.reshape(1, D), M2, mb2.reshape(1, 1))


def kernel(x, edge_index, W0, b0, W1, b1, W2, b2, M0, mb0, M1, mb1, M2, mb2):
    src = edge_index[0]
    dst = edge_index[1]
    pad = EPAD - E
    srcp = jnp.concatenate([src, jnp.zeros((pad,), jnp.int32)])
    dstp = jnp.concatenate([dst, jnp.full((pad,), N, jnp.int32)])
    zrow = jnp.zeros((B, D), _f32)
    onesr = jnp.ones((B, D), _f32)

    DG0, DG1 = _deg(dstp, zrow, onesr)
    A0, A1 = _segsum(x, srcp, dstp, zrow)
    h = _conv_tc(A0, A1, DG0, DG1, W0, b0)
    A0, A1 = _segsum(h, srcp, dstp, zrow)
    h = _conv_tc(A0, A1, DG0, DG1, W1, b1)
    A0, A1 = _segsum(h, srcp, dstp, zrow)
    return _final_tc(A0, A1, DG0, DG1, W2, b2, M0, mb0, M1, mb1, M2, mb2)
